# baseline (speedup 1.0000x reference)
import jax
import jax.numpy as jnp
from jax import lax
from jax.experimental import pallas as pl
from jax.experimental.pallas import tpu as pltpu

T = 2048
D = 1024
E = 8
FFN = 512
TOPK = 2
NGROUP = 4
EPG = E // NGROUP
SFFN = 512
SCALE = 2.5

TM = 256


def _silu(x):
    return x * jax.nn.sigmoid(x)


def _routing(scores, sb, gsum):
    eidx = lax.broadcasted_iota(jnp.int32, (TM, E), 1)
    gid = eidx // EPG
    giota = lax.broadcasted_iota(jnp.int32, (TM, NGROUP), 1)

    neg = jnp.float32(-jnp.inf)
    big = jnp.int32(NGROUP)

    m1 = jnp.max(gsum, axis=1, keepdims=True)
    g1 = jnp.min(jnp.where(gsum == m1, giota, big), axis=1, keepdims=True)
    gsum2 = jnp.where(giota == g1, neg, gsum)
    m2 = jnp.max(gsum2, axis=1, keepdims=True)
    g2 = jnp.min(jnp.where(gsum2 == m2, giota, big), axis=1, keepdims=True)

    group_ok = (gid == g1) | (gid == g2)
    tmp = jnp.where(group_ok, sb, jnp.float32(0.0))

    ebig = jnp.int32(E)
    t1 = jnp.max(tmp, axis=1, keepdims=True)
    e1 = jnp.min(jnp.where(tmp == t1, eidx, ebig), axis=1, keepdims=True)
    tmp2 = jnp.where(eidx == e1, neg, tmp)
    t2 = jnp.max(tmp2, axis=1, keepdims=True)
    e2 = jnp.min(jnp.where(tmp2 == t2, eidx, ebig), axis=1, keepdims=True)

    sel1 = eidx == e1
    sel2 = eidx == e2
    w1 = jnp.sum(jnp.where(sel1, scores, 0.0), axis=1, keepdims=True)
    w2 = jnp.sum(jnp.where(sel2, scores, 0.0), axis=1, keepdims=True)
    denom = w1 + w2
    combine = (jnp.where(sel1, w1, 0.0) + jnp.where(sel2, w2, 0.0)) / denom
    return combine


def _moe_body(x_ref, scores_ref, sb_ref, gsum_ref, wgu_ref, wd_ref, sgu_ref,
              sd_ref, out_ref):
    x = x_ref[...]

    combine = _routing(scores_ref[...], sb_ref[...], gsum_ref[...])

    gu = lax.dot_general(x, sgu_ref[...], (((1,), (1,)), ((), ())),
                         preferred_element_type=jnp.float32)
    h = _silu(gu[:, :SFFN]) * gu[:, SFFN:]
    shared = lax.dot_general(h, sd_ref[...], (((1,), (1,)), ((), ())),
                             preferred_element_type=jnp.float32)

    acc = jnp.zeros((TM, D), jnp.float32)
    for e in range(E):
        gue = lax.dot_general(x, wgu_ref[e], (((1,), (1,)), ((), ())),
                              preferred_element_type=jnp.float32)
        he = _silu(gue[:, :FFN]) * gue[:, FFN:]
        ye = lax.dot_general(he, wd_ref[e], (((1,), (1,)), ((), ())),
                             preferred_element_type=jnp.float32)
        acc = acc + combine[:, e:e + 1] * ye

    out_ref[...] = acc * SCALE + shared


@jax.jit
def _moe(hidden_states, scores, sb, gsum, w_gate_up, w_down, s_gate_up,
         s_down):
    nt = T // TM
    return pl.pallas_call(
        _moe_body,
        grid=(nt,),
        in_specs=[
            pl.BlockSpec((TM, D), lambda t: (t, 0)),
            pl.BlockSpec((TM, E), lambda t: (t, 0)),
            pl.BlockSpec((TM, E), lambda t: (t, 0)),
            pl.BlockSpec((TM, NGROUP), lambda t: (t, 0)),
            pl.BlockSpec((E, 2 * FFN, D), lambda t: (0, 0, 0)),
            pl.BlockSpec((E, D, FFN), lambda t: (0, 0, 0)),
            pl.BlockSpec((2 * SFFN, D), lambda t: (0, 0)),
            pl.BlockSpec((D, SFFN), lambda t: (0, 0)),
        ],
        out_specs=pl.BlockSpec((TM, D), lambda t: (t, 0)),
        out_shape=jax.ShapeDtypeStruct((T, D), jnp.float32),
        compiler_params=pltpu.CompilerParams(
            vmem_limit_bytes=100 * 1024 * 1024),
    )(hidden_states, scores, sb, gsum, w_gate_up, w_down, s_gate_up, s_down)


def kernel(hidden_states, gate_w, corr_bias, w_gate_up, w_down, s_gate_up,
           s_down):
    router_logits = hidden_states.astype(jnp.float32) @ gate_w.T
    scores = jax.nn.sigmoid(router_logits)
    sb = scores + corr_bias[None, :]
    gsum = lax.top_k(sb.reshape(T, NGROUP, EPG), 2)[0].sum(axis=-1)
    return _moe(hidden_states, scores, sb, gsum, w_gate_up, w_down,
                s_gate_up, s_down)
